# Initial kernel scaffold; baseline (speedup 1.0000x reference)
import jax
import jax.numpy as jnp
from jax.experimental import pallas as pl


def kernel(time_value, bin_edges, embed_table):
    raise NotImplementedError("write your pallas kernel here")


# speedup vs baseline: 24.9419x; 24.9419x over previous
import jax
import jax.numpy as jnp
from jax import lax
from jax.experimental import pallas as pl
from jax.experimental.pallas import tpu as pltpu
from jax.experimental.pallas import tpu_sc as plsc

N = 131072
DIM = 256
NUM_EDGES = 101
EDGE_PAD = 128
NC, NS, L = 2, 16, 16
NW = NC * NS
BPW = N // NW
CH = 128
NCH = BPW // CH


def _sc_body(time_hbm, edges_hbm, table_hbm, out_hbm,
             tv, ev, idxv, buf0, buf1, sem0, sem1):
    wid = lax.axis_index("s") * NC + lax.axis_index("c")
    base = wid * BPW
    pltpu.sync_copy(time_hbm.at[pl.ds(base, BPW)], tv)
    pltpu.sync_copy(edges_hbm, ev)

    def search_step(i, carry):
        t = tv[pl.ds(i * L, L)]
        pos = jnp.zeros((L,), jnp.int32)
        for s in (64, 32, 16, 8, 4, 2, 1):
            cand = pos + s
            e = plsc.load_gather(ev, [cand - 1])
            pos = jnp.where(e < t, cand, pos)
        idxv[pl.ds(i * L, L)] = jnp.minimum(pos, NUM_EDGES - 1)
        return carry

    lax.fori_loop(0, BPW // L, search_step, 0)

    def fire(c, buf, sem):
        return pltpu.async_copy(
            table_hbm.at[idxv.at[pl.ds(c * CH, CH)]], buf, sem)

    d0 = fire(0, buf0, sem0)
    d1 = None
    for k in range(0, NCH, 2):
        d1 = fire(k + 1, buf1, sem1)
        d0.wait()
        pltpu.sync_copy(buf0, out_hbm.at[pl.ds(base + k * CH, CH)])
        if k + 2 < NCH:
            d0 = fire(k + 2, buf0, sem0)
        d1.wait()
        pltpu.sync_copy(buf1, out_hbm.at[pl.ds(base + (k + 1) * CH, CH)])


def _build():
    mesh = plsc.VectorSubcoreMesh(core_axis_name="c", subcore_axis_name="s")
    return pl.kernel(
        _sc_body,
        out_type=jax.ShapeDtypeStruct((N, DIM), jnp.float32),
        mesh=mesh,
        compiler_params=pltpu.CompilerParams(needs_layout_passes=False),
        scratch_types=[
            pltpu.VMEM((BPW,), jnp.float32),
            pltpu.VMEM((EDGE_PAD,), jnp.float32),
            pltpu.VMEM((BPW,), jnp.int32),
            pltpu.VMEM((CH, DIM), jnp.float32),
            pltpu.VMEM((CH, DIM), jnp.float32),
            pltpu.SemaphoreType.DMA,
            pltpu.SemaphoreType.DMA,
        ],
    )


def _impl(time_value, bin_edges, embed_table):
    pad = jnp.full((EDGE_PAD - NUM_EDGES,), jnp.inf, dtype=jnp.float32)
    edges_pad = jnp.concatenate([bin_edges.astype(jnp.float32), pad])
    return _build()(time_value, edges_pad, embed_table)


_jitted = jax.jit(_impl)


def kernel(time_value, bin_edges, embed_table):
    return _jitted(time_value, bin_edges, embed_table)


# speedup vs baseline: 29.0603x; 1.1651x over previous
import jax
import jax.numpy as jnp
from jax import lax
from jax.experimental import pallas as pl
from jax.experimental.pallas import tpu as pltpu
from jax.experimental.pallas import tpu_sc as plsc

N = 131072
DIM = 256
NUM_EDGES = 101
EDGE_PAD = 128
NC, NS, L = 2, 16, 16
NW = NC * NS
BPW = N // NW
CH = 128
NCH = BPW // CH


def _sc_body(time_hbm, edges_hbm, table_hbm, out_hbm,
             tv, ev, tab, idxv, buf0, buf1, sem0, sem1):
    wid = lax.axis_index("s") * NC + lax.axis_index("c")
    base = wid * BPW
    pltpu.sync_copy(time_hbm.at[pl.ds(base, BPW)], tv)
    pltpu.sync_copy(edges_hbm, ev)
    pltpu.sync_copy(table_hbm, tab)

    def search_step(i, carry):
        t = tv[pl.ds(i * L, L)]
        pos = jnp.zeros((L,), jnp.int32)
        for s in (64, 32, 16, 8, 4, 2, 1):
            cand = pos + s
            e = plsc.load_gather(ev, [cand - 1])
            pos = jnp.where(e < t, cand, pos)
        idxv[pl.ds(i * L, L)] = jnp.minimum(pos, NUM_EDGES - 1)
        return carry

    lax.fori_loop(0, BPW // L, search_step, 0)

    def build(c, buf):
        def group_step(q, carry):
            iv = idxv[pl.ds(c * CH + q * L, L)]
            for l in range(L):
                i = iv[l]
                r = q * L + l
                for g in range(DIM // L):
                    buf[r, pl.ds(g * L, L)] = tab[i, pl.ds(g * L, L)]
            return carry
        lax.fori_loop(0, CH // L, group_step, 0)

    def fire(c, buf, sem):
        return pltpu.async_copy(buf, out_hbm.at[pl.ds(base + c * CH, CH)],
                                sem)

    def loop_body(k, carry):
        a = 2 * k
        b = 2 * k + 1

        @pl.when(k > 0)
        def _():
            pltpu.make_async_copy(
                buf0, out_hbm.at[pl.ds(base, CH)], sem0).wait()
        build(a, buf0)
        fire(a, buf0, sem0)

        @pl.when(k > 0)
        def _():
            pltpu.make_async_copy(
                buf1, out_hbm.at[pl.ds(base, CH)], sem1).wait()
        build(b, buf1)
        fire(b, buf1, sem1)
        return carry

    lax.fori_loop(0, NCH // 2, loop_body, 0)
    pltpu.make_async_copy(buf0, out_hbm.at[pl.ds(base, CH)], sem0).wait()
    pltpu.make_async_copy(buf1, out_hbm.at[pl.ds(base, CH)], sem1).wait()


def _build():
    mesh = plsc.VectorSubcoreMesh(core_axis_name="c", subcore_axis_name="s")
    return pl.kernel(
        _sc_body,
        out_type=jax.ShapeDtypeStruct((N, DIM), jnp.float32),
        mesh=mesh,
        compiler_params=pltpu.CompilerParams(needs_layout_passes=False),
        scratch_types=[
            pltpu.VMEM((BPW,), jnp.float32),
            pltpu.VMEM((EDGE_PAD,), jnp.float32),
            pltpu.VMEM((NUM_EDGES, DIM), jnp.float32),
            pltpu.VMEM((BPW,), jnp.int32),
            pltpu.VMEM((CH, DIM), jnp.float32),
            pltpu.VMEM((CH, DIM), jnp.float32),
            pltpu.SemaphoreType.DMA,
            pltpu.SemaphoreType.DMA,
        ],
    )


def _impl(time_value, bin_edges, embed_table):
    pad = jnp.full((EDGE_PAD - NUM_EDGES,), jnp.inf, dtype=jnp.float32)
    edges_pad = jnp.concatenate([bin_edges.astype(jnp.float32), pad])
    return _build()(time_value, edges_pad, embed_table)


_jitted = jax.jit(_impl)


def kernel(time_value, bin_edges, embed_table):
    return _jitted(time_value, bin_edges, embed_table)
